# baseline (speedup 1.0000x reference)
import jax
import jax.numpy as jnp
from jax import lax
from jax.experimental import pallas as pl
from jax.experimental.pallas import tpu as pltpu
from jax.experimental.pallas import tpu_sc as plsc

N = 10000
D = 128
E = 320000
NC = 2
NS = 16
NW = NC * NS
EB = 112
NB = -(-E // EB)
T = -(-NB // NW)
NB = T * NW
EP = NB * EB
EPT = E // NW
NP = 10240
RP = NP // NS
NBUF = 3


def _sc_mesh():
    return plsc.VectorSubcoreMesh(core_axis_name="c", subcore_axis_name="s",
                                  num_cores=NC, num_subcores=NS)


def _deg_body(dst_hbm, out_hbm, hist_v, dbuf_v):
    c = lax.axis_index("c")
    s = lax.axis_index("s")
    w = s * NC + c
    zero16 = jnp.zeros((16,), jnp.float32)

    @pl.loop(0, N // 16)
    def _zero(i):
        hist_v[pl.ds(i * 16, 16)] = zero16

    pltpu.sync_copy(dst_hbm.at[pl.ds(w * EPT, EPT)], dbuf_v)
    ones16 = jnp.ones((16,), jnp.float32)

    @pl.loop(0, EPT // 16)
    def _scat(i):
        idx = dbuf_v[pl.ds(i * 16, 16)]
        plsc.addupdate_scatter(hist_v, [idx], ones16)

    pltpu.sync_copy(hist_v, out_hbm.at[w, 0])


def _make_deg(interpret=False):
    return pl.kernel(
        _deg_body,
        out_type=jax.ShapeDtypeStruct((NW, 1, N), jnp.float32),
        mesh=_sc_mesh(),
        scratch_types=[
            pltpu.VMEM((N,), jnp.float32),
            pltpu.VMEM((EPT,), jnp.int32),
        ],
        compiler_params=pltpu.CompilerParams(needs_layout_passes=False),
        interpret=interpret,
    )


def _segsum_body(xs_hbm, eidx_hbm, zeros_hbm, out_hbm,
                 acc, ibuf0, ibuf1, ibuf2,
                 rows0, rows1, rows2,
                 gsem0, gsem1, gsem2,
                 ssem0, ssem1, ssem2):
    c = lax.axis_index("c")
    s = lax.axis_index("s")
    w = s * NC + c
    bufs = [(ibuf0, rows0, gsem0, ssem0), (ibuf1, rows1, gsem1, ssem1),
            (ibuf2, rows2, gsem2, ssem2)]

    pltpu.sync_copy(zeros_hbm, acc.at[pl.ds(s * RP, RP)])
    plsc.subcore_barrier()

    def _stage(j, b):
        ib, rw, gs, _ = bufs[b]
        pltpu.sync_copy(eidx_hbm.at[w + j * NW], ib)
        pltpu.async_copy(xs_hbm.at[ib.at[0]], rw, gs)

    def _step(j, b, refill):
        ib, rw, gs, ss = bufs[b]
        ibp, rwp, gsp, ssp = bufs[(b - 1) % NBUF]
        pltpu.make_async_copy(xs_hbm.at[ib.at[0]], rw, gs).wait()
        pltpu.async_copy(rw, acc.at[ib.at[1]], ss, add=True)
        if refill:
            pltpu.make_async_copy(rwp, acc.at[ibp.at[1]], ssp).wait()
            pltpu.sync_copy(eidx_hbm.at[w + (j + 2) * NW], ibp)
            pltpu.async_copy(xs_hbm.at[ibp.at[0]], rwp, gsp)

    for b in range(NBUF):
        _stage(b, b)
    _step(0, 0, False)

    @pl.loop(0, (T - 3) // NBUF)
    def _grp(g):
        j = NBUF * g + 1
        for k in range(NBUF):
            _step(j + k, (1 + k) % NBUF, True)

    for j in range(T - 2, T):
        _step(j, j % NBUF, False)
    for j in range(T - 3, T):
        b = j % NBUF
        pltpu.make_async_copy(bufs[b][1], acc.at[bufs[b][0].at[1]],
                              bufs[b][3]).wait()

    plsc.subcore_barrier()
    pltpu.sync_copy(acc.at[pl.ds(s * RP, RP)],
                    out_hbm.at[c, pl.ds(s * RP, RP)])


def _make_segsum(interpret=False):
    scratch = [pltpu.VMEM_SHARED((NP, D), jnp.float32)]
    scratch += [pltpu.VMEM((2, EB), jnp.int32) for _ in range(NBUF)]
    scratch += [pltpu.VMEM((EB, D), jnp.float32) for _ in range(NBUF)]
    scratch += [pltpu.SemaphoreType.DMA for _ in range(2 * NBUF)]
    return pl.kernel(
        _segsum_body,
        out_type=jax.ShapeDtypeStruct((NC, NP, D), jnp.float32),
        mesh=_sc_mesh(),
        scratch_types=scratch,
        interpret=interpret,
    )


_lazy = {}


def _deg_hist(*args):
    if "deg" not in _lazy:
        _lazy["deg"] = _make_deg()
    return _lazy["deg"](*args)


def _segsum(*args):
    if "seg" not in _lazy:
        _lazy["seg"] = _make_segsum()
    return _lazy["seg"](*args)


def _mm_body(x_ref, w_ref, o_ref):
    o_ref[...] = jnp.dot(x_ref[...], w_ref[...],
                         preferred_element_type=jnp.float32)


def _matmul(x, w):
    return pl.pallas_call(
        _mm_body,
        out_shape=jax.ShapeDtypeStruct((x.shape[0], w.shape[1]), jnp.float32),
    )(x, w)


def _prep_body(hist_ref, xw_ref, xs_ref, dinv_ref):
    ones = jnp.ones((NW, 1), jnp.float32)
    deg_col = lax.dot_general(hist_ref[...], ones, (((0,), (0,)), ((), ())),
                              precision=lax.Precision.HIGHEST)
    dinv = lax.rsqrt(deg_col + 1.0)
    dinv_ref[...] = dinv
    xs_ref[...] = xw_ref[...] * dinv


def _prep(hist, xw):
    return pl.pallas_call(
        _prep_body,
        out_shape=(jax.ShapeDtypeStruct((N, D), jnp.float32),
                   jax.ShapeDtypeStruct((N, 1), jnp.float32)),
    )(hist, xw)


def _mid_body(p_ref, xs_ref, dinv_ref, b1_ref, g1_ref, be1_ref, w2_ref,
              xs2_ref):
    dinv = dinv_ref[...]
    ps = lax.slice(p_ref[0] + p_ref[1], (0, 0), (N, D))
    h = (ps + xs_ref[...]) * dinv + b1_ref[...]
    mean = jnp.mean(h, axis=0)
    hc = h - mean
    var = jnp.mean(hc * hc, axis=0)
    h = hc * lax.rsqrt(var + 1e-5) * g1_ref[...] + be1_ref[...]
    h = jnp.maximum(h, 0.0)
    xs2_ref[...] = jnp.dot(h, w2_ref[...],
                           preferred_element_type=jnp.float32) * dinv


def _mid(p1, xs1, dinv, b1, g1, be1, W2):
    return pl.pallas_call(
        _mid_body,
        out_shape=jax.ShapeDtypeStruct((N, D), jnp.float32),
    )(p1, xs1, dinv, b1, g1, be1, W2)


def _fin_body(p_ref, xs2_ref, dinv_ref, b2_ref, g2_ref, be2_ref, o_ref):
    ps = lax.slice(p_ref[0] + p_ref[1], (0, 0), (N, D))
    h = (ps + xs2_ref[...]) * dinv_ref[...] + b2_ref[...]
    mean = jnp.mean(h, axis=0)
    hc = h - mean
    var = jnp.mean(hc * hc, axis=0)
    o_ref[...] = hc * lax.rsqrt(var + 1e-5) * g2_ref[...] + be2_ref[...]


def _fin(p2, xs2, dinv, b2, g2, be2):
    return pl.pallas_call(
        _fin_body,
        out_shape=jax.ShapeDtypeStruct((N, D), jnp.float32),
    )(p2, xs2, dinv, b2, g2, be2)


@jax.jit
def kernel(x, edge_index, W1, b1, g1, be1, W2, b2, g2, be2):
    ei = edge_index.astype(jnp.int32)
    pad = EP - E
    srcp = jnp.concatenate([ei[0], jnp.zeros((pad,), jnp.int32)])
    dstp = jnp.concatenate([ei[1], jnp.full((pad,), N, jnp.int32)])
    eidx = jnp.stack([srcp.reshape(NB, EB), dstp.reshape(NB, EB)], axis=1)
    zeros = jnp.zeros((RP, D), jnp.float32)

    hist = _deg_hist(ei[1]).reshape(NW, N)
    xw = _matmul(x, W1)
    xs1, dinv = _prep(hist, xw)
    p1 = _segsum(xs1, eidx, zeros)
    xs2 = _mid(p1, xs1, dinv, b1, g1, be1, W2)
    p2 = _segsum(xs2, eidx, zeros)
    return _fin(p2, xs2, dinv, b2, g2, be2)


# speedup vs baseline: 1.2872x; 1.2872x over previous
import jax
import jax.numpy as jnp
from jax import lax
from jax.experimental import pallas as pl
from jax.experimental.pallas import tpu as pltpu
from jax.experimental.pallas import tpu_sc as plsc

N = 10000
D = 128
E = 320000
NC = 2
NS = 16
NW = NC * NS
EB = 128
NB = E // EB
T = NB // NW
TAIL = NB - NW * T
EPT = E // NW
NP = 10240
RP = NP // NS


def _sc_mesh():
    return plsc.VectorSubcoreMesh(core_axis_name="c", subcore_axis_name="s",
                                  num_cores=NC, num_subcores=NS)


def _deg_body(dst_hbm, out_hbm, hist_v, dbuf_v):
    c = lax.axis_index("c")
    s = lax.axis_index("s")
    w = s * NC + c
    zero16 = jnp.zeros((16,), jnp.float32)

    @pl.loop(0, N // 16)
    def _zero(i):
        hist_v[pl.ds(i * 16, 16)] = zero16

    pltpu.sync_copy(dst_hbm.at[pl.ds(w * EPT, EPT)], dbuf_v)
    ones16 = jnp.ones((16,), jnp.float32)

    @pl.loop(0, EPT // 16)
    def _scat(i):
        idx = dbuf_v[pl.ds(i * 16, 16)]
        plsc.addupdate_scatter(hist_v, [idx], ones16)

    pltpu.sync_copy(hist_v, out_hbm.at[w, 0])


def _make_deg(interpret=False):
    return pl.kernel(
        _deg_body,
        out_type=jax.ShapeDtypeStruct((NW, 1, N), jnp.float32),
        mesh=_sc_mesh(),
        scratch_types=[
            pltpu.VMEM((N,), jnp.float32),
            pltpu.VMEM((EPT,), jnp.int32),
        ],
        compiler_params=pltpu.CompilerParams(needs_layout_passes=False),
        interpret=interpret,
    )


def _segsum_body(xs_hbm, eidx_hbm, zeros_hbm, out_hbm,
                 acc, ibuf0, ibuf1, ibuf2, ibuf3,
                 rows0, rows1, gsem0, gsem1, ssem0, ssem1):
    c = lax.axis_index("c")
    s = lax.axis_index("s")
    w = s * NC + c
    ibufs = [ibuf0, ibuf1, ibuf2, ibuf3]
    rows = [rows0, rows1]
    gsems = [gsem0, gsem1]
    ssems = [ssem0, ssem1]

    pltpu.sync_copy(zeros_hbm, acc.at[pl.ds(s * RP, RP)])
    plsc.subcore_barrier()

    def _idx(j, q):
        pltpu.sync_copy(eidx_hbm.at[w + j * NW], ibufs[q])

    def _gather(q, r):
        pltpu.async_copy(xs_hbm.at[ibufs[q].at[0]], rows[r], gsems[r])

    def _gwait(q, r):
        pltpu.make_async_copy(xs_hbm.at[ibufs[q].at[0]], rows[r],
                              gsems[r]).wait()

    def _scat(q, r):
        pltpu.async_copy(rows[r], acc.at[ibufs[q].at[1]], ssems[r], add=True)

    def _swait(q, r):
        pltpu.make_async_copy(rows[r], acc.at[ibufs[q].at[1]],
                              ssems[r]).wait()

    _idx(0, 0)
    _gather(0, 0)
    _idx(1, 1)
    _gather(1, 1)

    @pl.loop(0, T // 4)
    def _quad(i):
        j0 = 4 * i
        _gwait(0, 0)
        _scat(0, 0)
        _idx(j0 + 2, 2)
        _gwait(1, 1)
        _scat(1, 1)
        _idx(j0 + 3, 3)
        _swait(0, 0)
        _gather(2, 0)
        _swait(1, 1)
        _gather(3, 1)
        _gwait(2, 0)
        _scat(2, 0)
        _idx(j0 + 4, 0)
        _gwait(3, 1)
        _scat(3, 1)
        _idx(j0 + 5, 1)
        _swait(2, 0)
        _gather(0, 0)
        _swait(3, 1)
        _gather(1, 1)

    _gwait(0, 0)
    _scat(0, 0)
    _gwait(1, 1)
    _scat(1, 1)
    _swait(0, 0)
    _swait(1, 1)

    @pl.when(w < TAIL)
    def _tail():
        pltpu.sync_copy(eidx_hbm.at[NW * T + w], ibuf0)
        pltpu.async_copy(xs_hbm.at[ibuf0.at[0]], rows0, gsem0).wait()
        pltpu.sync_copy(rows0, acc.at[ibuf0.at[1]], add=True)

    plsc.subcore_barrier()
    pltpu.sync_copy(acc.at[pl.ds(s * RP, RP)],
                    out_hbm.at[c, pl.ds(s * RP, RP)])


def _make_segsum(interpret=False):
    scratch = [pltpu.VMEM_SHARED((NP, D), jnp.float32)]
    scratch += [pltpu.VMEM((2, EB), jnp.int32) for _ in range(4)]
    scratch += [pltpu.VMEM((EB, D), jnp.float32) for _ in range(2)]
    scratch += [pltpu.SemaphoreType.DMA for _ in range(4)]
    return pl.kernel(
        _segsum_body,
        out_type=jax.ShapeDtypeStruct((NC, NP, D), jnp.float32),
        mesh=_sc_mesh(),
        scratch_types=scratch,
        interpret=interpret,
    )


_lazy = {}


def _deg_hist(*args):
    if "deg" not in _lazy:
        _lazy["deg"] = _make_deg()
    return _lazy["deg"](*args)


def _segsum(*args):
    if "seg" not in _lazy:
        _lazy["seg"] = _make_segsum()
    return _lazy["seg"](*args)


def _mm_body(x_ref, w_ref, o_ref):
    o_ref[...] = jnp.dot(x_ref[...], w_ref[...],
                         preferred_element_type=jnp.float32)


def _matmul(x, w):
    return pl.pallas_call(
        _mm_body,
        out_shape=jax.ShapeDtypeStruct((x.shape[0], w.shape[1]), jnp.float32),
    )(x, w)


def _prep_body(hist_ref, xw_ref, xs_ref, dinv_ref):
    ones = jnp.ones((NW, 1), jnp.float32)
    deg_col = lax.dot_general(hist_ref[...], ones, (((0,), (0,)), ((), ())),
                              precision=lax.Precision.HIGHEST)
    dinv = lax.rsqrt(deg_col + 1.0)
    dinv_ref[...] = dinv
    xs_ref[...] = xw_ref[...] * dinv


def _prep(hist, xw):
    return pl.pallas_call(
        _prep_body,
        out_shape=(jax.ShapeDtypeStruct((N, D), jnp.float32),
                   jax.ShapeDtypeStruct((N, 1), jnp.float32)),
    )(hist, xw)


def _mid_body(p_ref, xs_ref, dinv_ref, b1_ref, g1_ref, be1_ref, w2_ref,
              xs2_ref):
    dinv = dinv_ref[...]
    ps = lax.slice(p_ref[0] + p_ref[1], (0, 0), (N, D))
    h = (ps + xs_ref[...]) * dinv + b1_ref[...]
    mean = jnp.mean(h, axis=0)
    hc = h - mean
    var = jnp.mean(hc * hc, axis=0)
    h = hc * lax.rsqrt(var + 1e-5) * g1_ref[...] + be1_ref[...]
    h = jnp.maximum(h, 0.0)
    xs2_ref[...] = jnp.dot(h, w2_ref[...],
                           preferred_element_type=jnp.float32) * dinv


def _mid(p1, xs1, dinv, b1, g1, be1, W2):
    return pl.pallas_call(
        _mid_body,
        out_shape=jax.ShapeDtypeStruct((N, D), jnp.float32),
    )(p1, xs1, dinv, b1, g1, be1, W2)


def _fin_body(p_ref, xs2_ref, dinv_ref, b2_ref, g2_ref, be2_ref, o_ref):
    ps = lax.slice(p_ref[0] + p_ref[1], (0, 0), (N, D))
    h = (ps + xs2_ref[...]) * dinv_ref[...] + b2_ref[...]
    mean = jnp.mean(h, axis=0)
    hc = h - mean
    var = jnp.mean(hc * hc, axis=0)
    o_ref[...] = hc * lax.rsqrt(var + 1e-5) * g2_ref[...] + be2_ref[...]


def _fin(p2, xs2, dinv, b2, g2, be2):
    return pl.pallas_call(
        _fin_body,
        out_shape=jax.ShapeDtypeStruct((N, D), jnp.float32),
    )(p2, xs2, dinv, b2, g2, be2)


@jax.jit
def kernel(x, edge_index, W1, b1, g1, be1, W2, b2, g2, be2):
    ei = edge_index.astype(jnp.int32)
    eidx = jnp.stack([ei[0].reshape(NB, EB), ei[1].reshape(NB, EB)], axis=1)
    zeros = jnp.zeros((RP, D), jnp.float32)

    hist = _deg_hist(ei[1]).reshape(NW, N)
    xw = _matmul(x, W1)
    xs1, dinv = _prep(hist, xw)
    p1 = _segsum(xs1, eidx, zeros)
    xs2 = _mid(p1, xs1, dinv, b1, g1, be1, W2)
    p2 = _segsum(xs2, eidx, zeros)
    return _fin(p2, xs2, dinv, b2, g2, be2)


# speedup vs baseline: 1.3849x; 1.0759x over previous
import jax
import jax.numpy as jnp
from jax import lax
from jax.experimental import pallas as pl
from jax.experimental.pallas import tpu as pltpu
from jax.experimental.pallas import tpu_sc as plsc

N = 10000
D = 128
E = 320000
NC = 2
NS = 16
NW = NC * NS
EB = 128
NB = E // EB
T = NB // NW
TAIL = NB - NW * T
EPT = E // NW
NP = 10240
RP = NP // NS


def _sc_mesh():
    return plsc.VectorSubcoreMesh(core_axis_name="c", subcore_axis_name="s",
                                  num_cores=NC, num_subcores=NS)


def _deg_body(dst_hbm, out_hbm, hist_v, dbuf_v):
    c = lax.axis_index("c")
    s = lax.axis_index("s")
    w = s * NC + c
    zero16 = jnp.zeros((16,), jnp.float32)

    @pl.loop(0, N // 16)
    def _zero(i):
        hist_v[pl.ds(i * 16, 16)] = zero16

    pltpu.sync_copy(dst_hbm.at[pl.ds(w * EPT, EPT)], dbuf_v)
    ones16 = jnp.ones((16,), jnp.float32)

    @pl.loop(0, EPT // 16)
    def _scat(i):
        idx = dbuf_v[pl.ds(i * 16, 16)]
        plsc.addupdate_scatter(hist_v, [idx], ones16)

    pltpu.sync_copy(hist_v, out_hbm.at[w, 0])


def _make_deg(interpret=False):
    return pl.kernel(
        _deg_body,
        out_type=jax.ShapeDtypeStruct((NW, 1, N), jnp.float32),
        mesh=_sc_mesh(),
        scratch_types=[
            pltpu.VMEM((N,), jnp.float32),
            pltpu.VMEM((EPT,), jnp.int32),
        ],
        compiler_params=pltpu.CompilerParams(needs_layout_passes=False),
        interpret=interpret,
    )


def _segsum_body(xs_hbm, eidx_hbm, zeros_hbm, out_hbm,
                 acc, ibuf0, ibuf1, rows0, rows1,
                 gsem0, gsem1, ssem0, ssem1):
    c = lax.axis_index("c")
    s = lax.axis_index("s")
    w = s * NC + c

    pltpu.sync_copy(zeros_hbm, acc.at[pl.ds(s * RP, RP)])
    plsc.subcore_barrier()

    pltpu.sync_copy(eidx_hbm.at[w], ibuf0)
    pltpu.async_copy(xs_hbm.at[ibuf0.at[0]], rows0, gsem0)
    pltpu.sync_copy(eidx_hbm.at[w + NW], ibuf1)
    pltpu.async_copy(xs_hbm.at[ibuf1.at[0]], rows1, gsem1)

    @pl.loop(0, T // 2 - 1)
    def _pair(i):
        j0 = 2 * i
        pltpu.make_async_copy(xs_hbm.at[ibuf0.at[0]], rows0, gsem0).wait()
        sc0 = pltpu.async_copy(rows0, acc.at[ibuf0.at[1]], ssem0, add=True)
        pltpu.make_async_copy(xs_hbm.at[ibuf1.at[0]], rows1, gsem1).wait()
        sc1 = pltpu.async_copy(rows1, acc.at[ibuf1.at[1]], ssem1, add=True)
        sc0.wait()
        pltpu.sync_copy(eidx_hbm.at[w + (j0 + 2) * NW], ibuf0)
        pltpu.async_copy(xs_hbm.at[ibuf0.at[0]], rows0, gsem0)
        sc1.wait()
        pltpu.sync_copy(eidx_hbm.at[w + (j0 + 3) * NW], ibuf1)
        pltpu.async_copy(xs_hbm.at[ibuf1.at[0]], rows1, gsem1)

    pltpu.make_async_copy(xs_hbm.at[ibuf0.at[0]], rows0, gsem0).wait()
    sc0 = pltpu.async_copy(rows0, acc.at[ibuf0.at[1]], ssem0, add=True)
    pltpu.make_async_copy(xs_hbm.at[ibuf1.at[0]], rows1, gsem1).wait()
    sc1 = pltpu.async_copy(rows1, acc.at[ibuf1.at[1]], ssem1, add=True)
    sc0.wait()
    sc1.wait()

    @pl.when(w < TAIL)
    def _tail():
        pltpu.sync_copy(eidx_hbm.at[NW * T + w], ibuf0)
        pltpu.async_copy(xs_hbm.at[ibuf0.at[0]], rows0, gsem0).wait()
        pltpu.sync_copy(rows0, acc.at[ibuf0.at[1]], add=True)

    plsc.subcore_barrier()
    pltpu.sync_copy(acc.at[pl.ds(s * RP, RP)],
                    out_hbm.at[c, pl.ds(s * RP, RP)])


def _make_segsum(interpret=False):
    return pl.kernel(
        _segsum_body,
        out_type=jax.ShapeDtypeStruct((NC, NP, D), jnp.float32),
        mesh=_sc_mesh(),
        scratch_types=[
            pltpu.VMEM_SHARED((NP, D), jnp.float32),
            pltpu.VMEM((2, EB), jnp.int32),
            pltpu.VMEM((2, EB), jnp.int32),
            pltpu.VMEM((EB, D), jnp.float32),
            pltpu.VMEM((EB, D), jnp.float32),
            pltpu.SemaphoreType.DMA,
            pltpu.SemaphoreType.DMA,
            pltpu.SemaphoreType.DMA,
            pltpu.SemaphoreType.DMA,
        ],
        interpret=interpret,
    )


_lazy = {}


def _deg_hist(*args):
    if "deg" not in _lazy:
        _lazy["deg"] = _make_deg()
    return _lazy["deg"](*args)


def _segsum(*args):
    if "seg" not in _lazy:
        _lazy["seg"] = _make_segsum()
    return _lazy["seg"](*args)


def _mm_body(x_ref, w_ref, o_ref):
    o_ref[...] = jnp.dot(x_ref[...], w_ref[...],
                         preferred_element_type=jnp.float32)


def _matmul(x, w):
    return pl.pallas_call(
        _mm_body,
        out_shape=jax.ShapeDtypeStruct((x.shape[0], w.shape[1]), jnp.float32),
    )(x, w)


def _prep_body(hist_ref, xw_ref, xs_ref, dinv_ref):
    ones = jnp.ones((NW, 1), jnp.float32)
    deg_col = lax.dot_general(hist_ref[...], ones, (((0,), (0,)), ((), ())),
                              precision=lax.Precision.HIGHEST)
    dinv = lax.rsqrt(deg_col + 1.0)
    dinv_ref[...] = dinv
    xs_ref[...] = xw_ref[...] * dinv


def _prep(hist, xw):
    return pl.pallas_call(
        _prep_body,
        out_shape=(jax.ShapeDtypeStruct((N, D), jnp.float32),
                   jax.ShapeDtypeStruct((N, 1), jnp.float32)),
    )(hist, xw)


def _mid_body(p_ref, xs_ref, dinv_ref, b1_ref, g1_ref, be1_ref, w2_ref,
              xs2_ref):
    dinv = dinv_ref[...]
    ps = lax.slice(p_ref[0] + p_ref[1], (0, 0), (N, D))
    h = (ps + xs_ref[...]) * dinv + b1_ref[...]
    mean = jnp.mean(h, axis=0)
    hc = h - mean
    var = jnp.mean(hc * hc, axis=0)
    h = hc * lax.rsqrt(var + 1e-5) * g1_ref[...] + be1_ref[...]
    h = jnp.maximum(h, 0.0)
    xs2_ref[...] = jnp.dot(h, w2_ref[...],
                           preferred_element_type=jnp.float32) * dinv


def _mid(p1, xs1, dinv, b1, g1, be1, W2):
    return pl.pallas_call(
        _mid_body,
        out_shape=jax.ShapeDtypeStruct((N, D), jnp.float32),
    )(p1, xs1, dinv, b1, g1, be1, W2)


def _fin_body(p_ref, xs2_ref, dinv_ref, b2_ref, g2_ref, be2_ref, o_ref):
    ps = lax.slice(p_ref[0] + p_ref[1], (0, 0), (N, D))
    h = (ps + xs2_ref[...]) * dinv_ref[...] + b2_ref[...]
    mean = jnp.mean(h, axis=0)
    hc = h - mean
    var = jnp.mean(hc * hc, axis=0)
    o_ref[...] = hc * lax.rsqrt(var + 1e-5) * g2_ref[...] + be2_ref[...]


def _fin(p2, xs2, dinv, b2, g2, be2):
    return pl.pallas_call(
        _fin_body,
        out_shape=jax.ShapeDtypeStruct((N, D), jnp.float32),
    )(p2, xs2, dinv, b2, g2, be2)


@jax.jit
def kernel(x, edge_index, W1, b1, g1, be1, W2, b2, g2, be2):
    ei = edge_index.astype(jnp.int32)
    eidx = jnp.stack([ei[0].reshape(NB, EB), ei[1].reshape(NB, EB)], axis=1)
    zeros = jnp.zeros((RP, D), jnp.float32)

    hist = _deg_hist(ei[1]).reshape(NW, N)
    xw = _matmul(x, W1)
    xs1, dinv = _prep(hist, xw)
    p1 = _segsum(xs1, eidx, zeros)
    xs2 = _mid(p1, xs1, dinv, b1, g1, be1, W2)
    p2 = _segsum(xs2, eidx, zeros)
    return _fin(p2, xs2, dinv, b2, g2, be2)


# speedup vs baseline: 1.3963x; 1.0082x over previous
import jax
import jax.numpy as jnp
from jax import lax
from jax.experimental import pallas as pl
from jax.experimental.pallas import tpu as pltpu
from jax.experimental.pallas import tpu_sc as plsc

N = 10000
D = 128
E = 320000
NC = 2
NS = 16
NW = NC * NS
EB = 128
NB = E // EB
T = NB // NW
TAIL = NB - NW * T
EPT = E // NW
NP = 10240
RP = NP // NS


def _sc_mesh():
    return plsc.VectorSubcoreMesh(core_axis_name="c", subcore_axis_name="s",
                                  num_cores=NC, num_subcores=NS)


def _deg_body(dst_hbm, out_hbm, hist_v, dbuf_v):
    c = lax.axis_index("c")
    s = lax.axis_index("s")
    w = s * NC + c
    zero16 = jnp.zeros((16,), jnp.float32)

    @pl.loop(0, N // 16)
    def _zero(i):
        hist_v[pl.ds(i * 16, 16)] = zero16

    pltpu.sync_copy(dst_hbm.at[pl.ds(w * EPT, EPT)], dbuf_v)
    ones16 = jnp.ones((16,), jnp.float32)

    @pl.loop(0, EPT // 16)
    def _scat(i):
        idx = dbuf_v[pl.ds(i * 16, 16)]
        plsc.addupdate_scatter(hist_v, [idx], ones16)

    pltpu.sync_copy(hist_v, out_hbm.at[w, 0])


def _make_deg(interpret=False):
    return pl.kernel(
        _deg_body,
        out_type=jax.ShapeDtypeStruct((NW, 1, N), jnp.float32),
        mesh=_sc_mesh(),
        scratch_types=[
            pltpu.VMEM((N,), jnp.float32),
            pltpu.VMEM((EPT,), jnp.int32),
        ],
        compiler_params=pltpu.CompilerParams(needs_layout_passes=False),
        interpret=interpret,
    )


def _segsum_body(xs_hbm, eidx_hbm, zeros_hbm, out_hbm,
                 acc, ibuf0, ibuf1, rows0, rows1,
                 gsem0, gsem1, ssem0, ssem1):
    c = lax.axis_index("c")
    s = lax.axis_index("s")
    w = s * NC + c

    pltpu.sync_copy(zeros_hbm, acc.at[pl.ds(s * RP, RP)])
    plsc.subcore_barrier()

    pltpu.sync_copy(eidx_hbm.at[w], ibuf0)
    pltpu.async_copy(xs_hbm.at[ibuf0.at[0]], rows0, gsem0)
    pltpu.sync_copy(eidx_hbm.at[w + NW], ibuf1)
    pltpu.async_copy(xs_hbm.at[ibuf1.at[0]], rows1, gsem1)

    @pl.loop(0, T // 2 - 1)
    def _pair(i):
        j0 = 2 * i
        pltpu.make_async_copy(xs_hbm.at[ibuf0.at[0]], rows0, gsem0).wait()
        sc0 = pltpu.async_copy(rows0, acc.at[ibuf0.at[1]], ssem0, add=True)
        pltpu.make_async_copy(xs_hbm.at[ibuf1.at[0]], rows1, gsem1).wait()
        sc1 = pltpu.async_copy(rows1, acc.at[ibuf1.at[1]], ssem1, add=True)
        sc0.wait()
        pltpu.sync_copy(eidx_hbm.at[w + (j0 + 2) * NW], ibuf0)
        pltpu.async_copy(xs_hbm.at[ibuf0.at[0]], rows0, gsem0)
        sc1.wait()
        pltpu.sync_copy(eidx_hbm.at[w + (j0 + 3) * NW], ibuf1)
        pltpu.async_copy(xs_hbm.at[ibuf1.at[0]], rows1, gsem1)

    pltpu.make_async_copy(xs_hbm.at[ibuf0.at[0]], rows0, gsem0).wait()
    sc0 = pltpu.async_copy(rows0, acc.at[ibuf0.at[1]], ssem0, add=True)
    pltpu.make_async_copy(xs_hbm.at[ibuf1.at[0]], rows1, gsem1).wait()
    sc1 = pltpu.async_copy(rows1, acc.at[ibuf1.at[1]], ssem1, add=True)
    sc0.wait()
    sc1.wait()

    @pl.when(w < TAIL)
    def _tail():
        pltpu.sync_copy(eidx_hbm.at[NW * T + w], ibuf0)
        pltpu.async_copy(xs_hbm.at[ibuf0.at[0]], rows0, gsem0).wait()
        pltpu.sync_copy(rows0, acc.at[ibuf0.at[1]], add=True)

    plsc.subcore_barrier()
    pltpu.sync_copy(acc.at[pl.ds(s * RP, RP)],
                    out_hbm.at[c, pl.ds(s * RP, RP)])


def _make_segsum(interpret=False):
    return pl.kernel(
        _segsum_body,
        out_type=jax.ShapeDtypeStruct((NC, NP, D), jnp.float32),
        mesh=_sc_mesh(),
        scratch_types=[
            pltpu.VMEM_SHARED((NP, D), jnp.float32),
            pltpu.VMEM((2, EB), jnp.int32),
            pltpu.VMEM((2, EB), jnp.int32),
            pltpu.VMEM((EB, D), jnp.float32),
            pltpu.VMEM((EB, D), jnp.float32),
            pltpu.SemaphoreType.DMA,
            pltpu.SemaphoreType.DMA,
            pltpu.SemaphoreType.DMA,
            pltpu.SemaphoreType.DMA,
        ],
        interpret=interpret,
    )


_lazy = {}


def _deg_hist(*args):
    if "deg" not in _lazy:
        _lazy["deg"] = _make_deg()
    return _lazy["deg"](*args)


def _segsum(*args):
    if "seg" not in _lazy:
        _lazy["seg"] = _make_segsum()
    return _lazy["seg"](*args)


def _mm_body(x_ref, w_ref, o_ref):
    o_ref[...] = jnp.dot(x_ref[...], w_ref[...],
                         preferred_element_type=jnp.float32)


def _matmul(x, w):
    return pl.pallas_call(
        _mm_body,
        out_shape=jax.ShapeDtypeStruct((x.shape[0], w.shape[1]), jnp.float32),
    )(x, w)


def _prep_body(hist_ref, x_ref, w1_ref, xs_ref, dinv_ref):
    ones = jnp.ones((NW, 1), jnp.float32)
    deg_col = lax.dot_general(hist_ref[...], ones, (((0,), (0,)), ((), ())),
                              precision=lax.Precision.HIGHEST)
    dinv = lax.rsqrt(deg_col + 1.0)
    dinv_ref[...] = dinv
    xw = jnp.dot(x_ref[...], w1_ref[...], preferred_element_type=jnp.float32)
    xs_ref[...] = xw * dinv


def _prep(hist, x, W1):
    return pl.pallas_call(
        _prep_body,
        out_shape=(jax.ShapeDtypeStruct((N, D), jnp.float32),
                   jax.ShapeDtypeStruct((N, 1), jnp.float32)),
    )(hist, x, W1)


def _mid_body(p_ref, xs_ref, dinv_ref, b1_ref, g1_ref, be1_ref, w2_ref,
              xs2_ref):
    dinv = dinv_ref[...]
    ps = lax.slice(p_ref[0] + p_ref[1], (0, 0), (N, D))
    h = (ps + xs_ref[...]) * dinv + b1_ref[...]
    mean = jnp.mean(h, axis=0)
    hc = h - mean
    var = jnp.mean(hc * hc, axis=0)
    h = hc * lax.rsqrt(var + 1e-5) * g1_ref[...] + be1_ref[...]
    h = jnp.maximum(h, 0.0)
    xs2_ref[...] = jnp.dot(h, w2_ref[...],
                           preferred_element_type=jnp.float32) * dinv


def _mid(p1, xs1, dinv, b1, g1, be1, W2):
    return pl.pallas_call(
        _mid_body,
        out_shape=jax.ShapeDtypeStruct((N, D), jnp.float32),
    )(p1, xs1, dinv, b1, g1, be1, W2)


def _fin_body(p_ref, xs2_ref, dinv_ref, b2_ref, g2_ref, be2_ref, o_ref):
    ps = lax.slice(p_ref[0] + p_ref[1], (0, 0), (N, D))
    h = (ps + xs2_ref[...]) * dinv_ref[...] + b2_ref[...]
    mean = jnp.mean(h, axis=0)
    hc = h - mean
    var = jnp.mean(hc * hc, axis=0)
    o_ref[...] = hc * lax.rsqrt(var + 1e-5) * g2_ref[...] + be2_ref[...]


def _fin(p2, xs2, dinv, b2, g2, be2):
    return pl.pallas_call(
        _fin_body,
        out_shape=jax.ShapeDtypeStruct((N, D), jnp.float32),
    )(p2, xs2, dinv, b2, g2, be2)


@jax.jit
def kernel(x, edge_index, W1, b1, g1, be1, W2, b2, g2, be2):
    ei = edge_index.astype(jnp.int32)
    eidx = jnp.stack([ei[0].reshape(NB, EB), ei[1].reshape(NB, EB)], axis=1)
    zeros = jnp.zeros((RP, D), jnp.float32)

    hist = _deg_hist(ei[1]).reshape(NW, N)
    xs1, dinv = _prep(hist, x, W1)
    p1 = _segsum(xs1, eidx, zeros)
    xs2 = _mid(p1, xs1, dinv, b1, g1, be1, W2)
    p2 = _segsum(xs2, eidx, zeros)
    return _fin(p2, xs2, dinv, b2, g2, be2)


# speedup vs baseline: 1.3981x; 1.0013x over previous
import jax
import jax.numpy as jnp
from jax import lax
from jax.experimental import pallas as pl
from jax.experimental.pallas import tpu as pltpu
from jax.experimental.pallas import tpu_sc as plsc

N = 10000
D = 128
E = 320000
NC = 2
NS = 16
NW = NC * NS
EB = 128
NB = E // EB
T = NB // NW
TAIL = NB - NW * T
EPT = E // NW
NP = 10240
RP = NP // NS


def _sc_mesh():
    return plsc.VectorSubcoreMesh(core_axis_name="c", subcore_axis_name="s",
                                  num_cores=NC, num_subcores=NS)


def _deg_body(dst_hbm, out_hbm, hist_v, dbuf_v):
    c = lax.axis_index("c")
    s = lax.axis_index("s")
    w = s * NC + c
    zero16 = jnp.zeros((16,), jnp.float32)

    @pl.loop(0, N // 16)
    def _zero(i):
        hist_v[pl.ds(i * 16, 16)] = zero16

    pltpu.sync_copy(dst_hbm.at[pl.ds(w * EPT, EPT)], dbuf_v)
    ones16 = jnp.ones((16,), jnp.float32)

    @pl.loop(0, EPT // 16)
    def _scat(i):
        idx = dbuf_v[pl.ds(i * 16, 16)]
        plsc.addupdate_scatter(hist_v, [idx], ones16)

    pltpu.sync_copy(hist_v, out_hbm.at[w, 0])


def _make_deg(interpret=False):
    return pl.kernel(
        _deg_body,
        out_type=jax.ShapeDtypeStruct((NW, 1, N), jnp.float32),
        mesh=_sc_mesh(),
        scratch_types=[
            pltpu.VMEM((N,), jnp.float32),
            pltpu.VMEM((EPT,), jnp.int32),
        ],
        compiler_params=pltpu.CompilerParams(needs_layout_passes=False),
        interpret=interpret,
    )


def _segsum_body(xs_hbm, eidx_hbm, zeros_hbm, out_hbm,
                 acc, ibuf0, ibuf1, rows0, rows1,
                 gsem0, gsem1, ssem0, ssem1):
    c = lax.axis_index("c")
    s = lax.axis_index("s")
    w = s * NC + c

    pltpu.sync_copy(zeros_hbm, acc.at[pl.ds(s * RP, RP)])
    plsc.subcore_barrier()

    pltpu.sync_copy(eidx_hbm.at[w], ibuf0)
    pltpu.async_copy(xs_hbm.at[ibuf0.at[0]], rows0, gsem0)
    pltpu.sync_copy(eidx_hbm.at[w + NW], ibuf1)
    pltpu.async_copy(xs_hbm.at[ibuf1.at[0]], rows1, gsem1)

    @pl.loop(0, T // 2 - 1)
    def _pair(i):
        j0 = 2 * i
        pltpu.make_async_copy(xs_hbm.at[ibuf0.at[0]], rows0, gsem0).wait()
        sc0 = pltpu.async_copy(rows0, acc.at[ibuf0.at[1]], ssem0, add=True)
        pltpu.make_async_copy(xs_hbm.at[ibuf1.at[0]], rows1, gsem1).wait()
        sc1 = pltpu.async_copy(rows1, acc.at[ibuf1.at[1]], ssem1, add=True)
        sc0.wait()
        pltpu.sync_copy(eidx_hbm.at[w + (j0 + 2) * NW], ibuf0)
        pltpu.async_copy(xs_hbm.at[ibuf0.at[0]], rows0, gsem0)
        sc1.wait()
        pltpu.sync_copy(eidx_hbm.at[w + (j0 + 3) * NW], ibuf1)
        pltpu.async_copy(xs_hbm.at[ibuf1.at[0]], rows1, gsem1)

    pltpu.make_async_copy(xs_hbm.at[ibuf0.at[0]], rows0, gsem0).wait()
    sc0 = pltpu.async_copy(rows0, acc.at[ibuf0.at[1]], ssem0, add=True)
    pltpu.make_async_copy(xs_hbm.at[ibuf1.at[0]], rows1, gsem1).wait()
    sc1 = pltpu.async_copy(rows1, acc.at[ibuf1.at[1]], ssem1, add=True)
    sc0.wait()
    sc1.wait()

    @pl.when(w < TAIL)
    def _tail():
        pltpu.sync_copy(eidx_hbm.at[NW * T + w], ibuf0)
        pltpu.async_copy(xs_hbm.at[ibuf0.at[0]], rows0, gsem0).wait()
        pltpu.sync_copy(rows0, acc.at[ibuf0.at[1]], add=True)

    plsc.subcore_barrier()
    pltpu.sync_copy(acc.at[pl.ds(s * RP, RP)],
                    out_hbm.at[c, pl.ds(s * RP, RP)])


def _make_segsum(interpret=False):
    return pl.kernel(
        _segsum_body,
        out_type=jax.ShapeDtypeStruct((NC, NP, D), jnp.float32),
        mesh=_sc_mesh(),
        scratch_types=[
            pltpu.VMEM_SHARED((NP, D), jnp.float32),
            pltpu.VMEM((2, EB), jnp.int32),
            pltpu.VMEM((2, EB), jnp.int32),
            pltpu.VMEM((EB, D), jnp.float32),
            pltpu.VMEM((EB, D), jnp.float32),
            pltpu.SemaphoreType.DMA,
            pltpu.SemaphoreType.DMA,
            pltpu.SemaphoreType.DMA,
            pltpu.SemaphoreType.DMA,
        ],
        interpret=interpret,
    )


_lazy = {}


def _deg_hist(*args):
    if "deg" not in _lazy:
        _lazy["deg"] = _make_deg()
    return _lazy["deg"](*args)


def _segsum(*args):
    if "seg" not in _lazy:
        _lazy["seg"] = _make_segsum()
    return _lazy["seg"](*args)


def _prep_body(hist_ref, x_ref, w1_ref, xs_ref, dinv_ref):
    ones = jnp.ones((NW, 1), jnp.float32)
    deg_col = lax.dot_general(hist_ref[...], ones, (((0,), (0,)), ((), ())),
                              precision=lax.Precision.HIGHEST)
    dinv = lax.rsqrt(deg_col + 1.0)
    dinv_ref[...] = dinv
    xw = jnp.dot(x_ref[...], w1_ref[...], preferred_element_type=jnp.float32)
    xs_ref[...] = xw * dinv


def _prep(hist, x, W1):
    return pl.pallas_call(
        _prep_body,
        out_shape=(jax.ShapeDtypeStruct((N, D), jnp.float32),
                   jax.ShapeDtypeStruct((N, 1), jnp.float32)),
    )(hist, x, W1)


def _mid_body(p_ref, xs_ref, dinv_ref, b1_ref, g1_ref, be1_ref, w2_ref,
              xs2_ref):
    dinv = dinv_ref[...]
    ps = lax.slice(p_ref[0] + p_ref[1], (0, 0), (N, D))
    h = (ps + xs_ref[...]) * dinv + b1_ref[...]
    mean = jnp.mean(h, axis=0)
    hc = h - mean
    var = jnp.mean(hc * hc, axis=0)
    h = hc * lax.rsqrt(var + 1e-5) * g1_ref[...] + be1_ref[...]
    h = jnp.maximum(h, 0.0)
    xs2_ref[...] = jnp.dot(h, w2_ref[...],
                           preferred_element_type=jnp.float32) * dinv


def _mid(p1, xs1, dinv, b1, g1, be1, W2):
    return pl.pallas_call(
        _mid_body,
        out_shape=jax.ShapeDtypeStruct((N, D), jnp.float32),
    )(p1, xs1, dinv, b1, g1, be1, W2)


def _fin_body(p_ref, xs2_ref, dinv_ref, b2_ref, g2_ref, be2_ref, o_ref):
    ps = lax.slice(p_ref[0] + p_ref[1], (0, 0), (N, D))
    h = (ps + xs2_ref[...]) * dinv_ref[...] + b2_ref[...]
    mean = jnp.mean(h, axis=0)
    hc = h - mean
    var = jnp.mean(hc * hc, axis=0)
    o_ref[...] = hc * lax.rsqrt(var + 1e-5) * g2_ref[...] + be2_ref[...]


def _fin(p2, xs2, dinv, b2, g2, be2):
    return pl.pallas_call(
        _fin_body,
        out_shape=jax.ShapeDtypeStruct((N, D), jnp.float32),
    )(p2, xs2, dinv, b2, g2, be2)


@jax.jit
def kernel(x, edge_index, W1, b1, g1, be1, W2, b2, g2, be2):
    ei = edge_index.astype(jnp.int32)
    eidx = jnp.stack([ei[0].reshape(NB, EB), ei[1].reshape(NB, EB)], axis=1)
    zeros = jnp.zeros((RP, D), jnp.float32)

    hist = _deg_hist(ei[1]).reshape(NW, N)
    xs1, dinv = _prep(hist, x, W1)
    p1 = _segsum(xs1, eidx, zeros)
    xs2 = _mid(p1, xs1, dinv, b1, g1, be1, W2)
    p2 = _segsum(xs2, eidx, zeros)
    return _fin(p2, xs2, dinv, b2, g2, be2)
